# baseline (speedup 1.0000x reference)
import functools

import jax
import jax.numpy as jnp
from jax import lax
from jax.experimental import pallas as pl
from jax.experimental.pallas import tpu as pltpu
from jax.experimental.pallas import tpu_sc as plsc

HID = 15
NSUB = 16
NCORE = 2
NW = NSUB * NCORE
CH = 128

LVL = (
    (10000, 10240, 327680, 80),
    (10000, 10240, 327680, 80),
    (5000, 5120, 163840, 40),
)
HEAD_EDGE = ((0, 0), (1, 2), (3, 3))
ZROWS = 10240 // NSUB
BLK = 640


def _f32(x):
    return x.astype(jnp.float32)


def _pad_rows(x, npad):
    return jnp.concatenate(
        [x, jnp.zeros((npad - x.shape[0], x.shape[1]), x.dtype)], axis=0)


def _head_weights(p, din_pad):
    Wl, bl, a = _f32(p["ld"]["W"]), _f32(p["ld"]["b"]), _f32(p["ld"]["a"])
    Wp, bp = _f32(p["p"]["W"]), _f32(p["p"]["b"])
    WS = jnp.concatenate([Wl, (Wl @ a[:HID])[:, None]], 1)
    bS = jnp.concatenate([bl, (bl @ a[:HID])[None]])
    WD = jnp.concatenate([Wp, (Wl @ a[HID:])[:, None]], 1)
    bD = jnp.concatenate([bp, (bl @ a[HID:])[None]])
    W = jnp.stack([WS, WD])
    if W.shape[1] < din_pad:
        W = jnp.concatenate(
            [W, jnp.zeros((2, din_pad - W.shape[1], 16), W.dtype)], 1)
    return W, jnp.stack([bS, bD])


def _stage_weights(params, stage, din_pad):
    Ws, bs = [], []
    for l in range(3):
        sp = params[f"lvl{l}"][stage]
        Wh = [_head_weights(sp[h], din_pad) for h in range(2)]
        Ws.append(jnp.stack([w for w, _ in Wh]))
        bs.append(jnp.stack([b for _, b in Wh]))
    return Ws, bs


def _tables_s1_body(x, w, b, *outs):
    xb = x[...]
    for h in range(2):
        for t in range(2):
            outs[h * 2 + t][...] = (
                jnp.dot(xb, w[h, t], preferred_element_type=jnp.float32)
                + b[h, t])


def _combine_block(num_ref, hp_ref):
    outs = []
    for h in range(2):
        num = num_ref[0, h] + num_ref[1, h]
        hp = hp_ref[0, h] + hp_ref[1, h]
        outs.append(2.0 * num[:, :HID] / (num[:, HID:HID + 1] + 1e-16)
                    + hp[:, :HID])
    xs = jax.nn.relu(jnp.concatenate(outs, 1))
    return jnp.concatenate(
        [xs, jnp.zeros((xs.shape[0], 2), xs.dtype)], 1)


def _combine_tables_body(num, hp, w, b, *outs):
    xsp = _combine_block(num, hp)
    outs[4][...] = xsp
    for h in range(2):
        for t in range(2):
            outs[h * 2 + t][...] = (
                jnp.dot(xsp, w[h, t], preferred_element_type=jnp.float32)
                + b[h, t])


def _pool_body(num, hp, xa, xb, bt, wf, bf, acc):
    i = pl.program_id(0)

    @pl.when(i == 0)
    def _():
        acc[...] = jnp.zeros_like(acc)

    xs3 = _combine_block(num, hp)
    xcat = jnp.concatenate([xa[...][:, :30], xb[...][:, :30], xs3[:, :30]], 1)
    x4 = jnp.dot(xcat, wf[...], preferred_element_type=jnp.float32) + bf[...]
    blk = x4.shape[0]
    x4e = jnp.concatenate(
        [x4, jnp.ones((blk, 1), jnp.float32), jnp.zeros((blk, 5), jnp.float32)],
        1)
    oh = (bt[0, 0, :][:, None]
          == lax.broadcasted_iota(jnp.int32, (blk, 64), 1)).astype(jnp.float32)
    acc[...] += lax.dot_general(oh, x4e, (((0,), (0,)), ((), ())),
                                preferred_element_type=jnp.float32)


def _comb_body(a0, a1, a2, wc, bc, out):
    pooled = []
    for a in (a0, a1, a2):
        acc = a[...]
        pooled.append(acc[:, :10] / jnp.maximum(acc[:, 10:11], 1.0))
    z = jnp.dot(jnp.concatenate(pooled, 1), wc[...],
                preferred_element_type=jnp.float32) + bc[...]
    z = z - jnp.max(z, axis=1, keepdims=True)
    e = jnp.exp(z)
    out[...] = e / jnp.sum(e, axis=1, keepdims=True)


def _sc_body(*refs):
    tabs = refs[0:12]
    edges = refs[12:24]
    zeros_hbm = refs[24]
    numouts = refs[25:28]
    hpouts = refs[28:31]
    (acc_num, acc_hp, srows0, drows0, srows1, drows1,
     srcidx0, dstidx0, valsv0, srcidx1, dstidx1, valsv1,
     ssr0, sdr0, ssi0, sdi0, ssr1, sdr1, ssi1, sdi1,
     sem_s0, sem_d0, sem_s1, sem_d1,
     sem_n0, sem_h0, sem_n1, sem_h1) = refs[31:]
    bufs = ((srcidx0, dstidx0, valsv0, srows0, drows0, sem_s0, sem_d0,
             ssr0, sdr0, ssi0, sdi0, sem_n0, sem_h0),
            (srcidx1, dstidx1, valsv1, srows1, drows1, sem_s1, sem_d1,
             ssr1, sdr1, ssi1, sdi1, sem_n1, sem_h1))

    c = lax.axis_index("c")
    s = lax.axis_index("s")
    wid = c * NSUB + s
    lane = lax.iota(jnp.int32, 16)
    is15 = lane == 15
    c15 = jnp.full((16,), 15, jnp.int32)

    for l, (n, npad, Epad, K) in enumerate(LVL):
        nr = npad // NSUB
        r0 = s * nr
        for h in range(2):
            S_hbm = tabs[l * 4 + h * 2 + 0]
            D_hbm = tabs[l * 4 + h * 2 + 1]
            e = HEAD_EDGE[l][h]
            src_h, dst_h, val_h = edges[e * 3], edges[e * 3 + 1], edges[e * 3 + 2]

            pltpu.sync_copy(zeros_hbm.at[pl.ds(0, nr)], acc_num.at[pl.ds(r0, nr)])
            pltpu.sync_copy(zeros_hbm.at[pl.ds(0, nr)], acc_hp.at[pl.ds(r0, nr)])
            plsc.subcore_barrier()

            def fetch_start(j, b, src_h=src_h, dst_h=dst_h, val_h=val_h,
                            S_hbm=S_hbm, D_hbm=D_hbm, K=K):
                si, di, vv, sr, dr, ss, sd = bufs[b][:7]
                off = wid * (K * CH) + j * CH
                pltpu.sync_copy(src_h.at[pl.ds(off, CH)], si)
                pltpu.sync_copy(dst_h.at[pl.ds(off, CH)], di)
                pltpu.sync_copy(val_h.at[pl.ds(off, CH)], vv)
                pltpu.async_copy(S_hbm.at[si], sr, ss)
                pltpu.async_copy(D_hbm.at[di], dr, sd)

            def process(j, b, S_hbm=S_hbm, D_hbm=D_hbm,
                        src_h=src_h, dst_h=dst_h, K=K):
                (si, di, vv_ref, sr, dr, ss, sd,
                 osr, odr, osi, odi, sn, sh) = bufs[b]
                off = wid * (K * CH) + j * CH
                pltpu.sync_copy(src_h.at[pl.ds(off, CH)], osi)
                pltpu.sync_copy(dst_h.at[pl.ds(off, CH)], odi)
                pltpu.make_async_copy(S_hbm.at[si], sr, ss).wait()
                pltpu.make_async_copy(D_hbm.at[di], dr, sd).wait()

                def grpbody(g, _):
                    vv = vv_ref[pl.ds(g * 16, 16)]
                    for jj in range(16):
                        k = g * 16 + jj
                        srow = sr[k]
                        drow = dr[k]
                        z = srow + drow
                        zb = z.at[c15].get(mode="promise_in_bounds")
                        ex = jnp.exp(jnp.maximum(zb, 0.2 * zb))
                        osr[k] = jnp.where(is15, ex, ex * srow)
                        vj = vv.at[jnp.full((16,), jj, jnp.int32)].get(
                            mode="promise_in_bounds")
                        odr[k] = vj * drow
                    return 0

                lax.fori_loop(0, CH // 16, grpbody, 0)
                pltpu.async_copy(osr, acc_num.at[odi], sn, add=True)
                pltpu.async_copy(odr, acc_hp.at[osi], sh, add=True)

            def wait_scat(b):
                (_, _, _, _, _, _, _,
                 osr, odr, osi, odi, sn, sh) = bufs[b]
                pltpu.make_async_copy(osr, acc_num.at[odi], sn).wait()
                pltpu.make_async_copy(odr, acc_hp.at[osi], sh).wait()

            fetch_start(0, 0)

            def pairbody(p, _, K=K, fetch_start=fetch_start, process=process):
                j0 = 2 * p

                @pl.when(p > 0)
                def _():
                    wait_scat(0)

                fetch_start(j0 + 1, 1)
                process(j0, 0)

                @pl.when(j0 + 2 < K)
                def _():
                    fetch_start(j0 + 2, 0)

                @pl.when(p > 0)
                def _():
                    wait_scat(1)

                process(j0 + 1, 1)
                return 0

            lax.fori_loop(0, K // 2, pairbody, 0)
            wait_scat(0)
            wait_scat(1)
            plsc.subcore_barrier()
            pltpu.sync_copy(acc_num.at[pl.ds(r0, nr)],
                            numouts[l].at[c, h, pl.ds(r0, nr)])
            pltpu.sync_copy(acc_hp.at[pl.ds(r0, nr)],
                            hpouts[l].at[c, h, pl.ds(r0, nr)])
            plsc.subcore_barrier()


def _make_sc_call():
    mesh = plsc.VectorSubcoreMesh(core_axis_name="c", subcore_axis_name="s")
    out_type = ([jax.ShapeDtypeStruct((2, 2, npad, 16), jnp.float32)
                 for (_, npad, _, _) in LVL]
                + [jax.ShapeDtypeStruct((2, 2, npad, 16), jnp.float32)
                   for (_, npad, _, _) in LVL])
    scratch = [
        pltpu.VMEM_SHARED((10240, 16), jnp.float32),
        pltpu.VMEM_SHARED((10240, 16), jnp.float32),
        pltpu.VMEM((CH, 16), jnp.float32),
        pltpu.VMEM((CH, 16), jnp.float32),
        pltpu.VMEM((CH, 16), jnp.float32),
        pltpu.VMEM((CH, 16), jnp.float32),
        pltpu.VMEM((CH,), jnp.int32),
        pltpu.VMEM((CH,), jnp.int32),
        pltpu.VMEM((CH,), jnp.float32),
        pltpu.VMEM((CH,), jnp.int32),
        pltpu.VMEM((CH,), jnp.int32),
        pltpu.VMEM((CH,), jnp.float32),
        pltpu.VMEM((CH, 16), jnp.float32),
        pltpu.VMEM((CH, 16), jnp.float32),
        pltpu.VMEM((CH,), jnp.int32),
        pltpu.VMEM((CH,), jnp.int32),
        pltpu.VMEM((CH, 16), jnp.float32),
        pltpu.VMEM((CH, 16), jnp.float32),
        pltpu.VMEM((CH,), jnp.int32),
        pltpu.VMEM((CH,), jnp.int32),
        pltpu.SemaphoreType.DMA,
        pltpu.SemaphoreType.DMA,
        pltpu.SemaphoreType.DMA,
        pltpu.SemaphoreType.DMA,
        pltpu.SemaphoreType.DMA,
        pltpu.SemaphoreType.DMA,
        pltpu.SemaphoreType.DMA,
        pltpu.SemaphoreType.DMA,
    ]
    return pl.kernel(_sc_body, out_type=out_type, mesh=mesh,
                     scratch_types=scratch,
                     compiler_params=pltpu.CompilerParams(
                         use_tc_tiling_on_sc=False))


_SC_CALL = None


def _sc_call():
    global _SC_CALL
    if _SC_CALL is None:
        _SC_CALL = _make_sc_call()
    return _SC_CALL


def _full_spec(shape):
    return pl.BlockSpec(shape, lambda i: tuple(0 for _ in shape))


def _row_spec(cols):
    return pl.BlockSpec((BLK, cols), lambda i: (i, 0))


def _acc_spec():
    return pl.BlockSpec((2, 2, BLK, 16), lambda i: (0, 0, i, 0))


def _tables_s1(xp, w, b, npad):
    grid = npad // BLK
    return pl.pallas_call(
        _tables_s1_body,
        grid=grid,
        in_specs=[_row_spec(128), _full_spec(w.shape), _full_spec(b.shape)],
        out_specs=[_row_spec(16)] * 4,
        out_shape=[jax.ShapeDtypeStruct((npad, 16), jnp.float32)] * 4,
    )(xp, w, b)


def _combine_tables(num, hp, w, b, npad):
    grid = npad // BLK
    outs = pl.pallas_call(
        _combine_tables_body,
        grid=grid,
        in_specs=[_acc_spec(), _acc_spec(),
                  _full_spec(w.shape), _full_spec(b.shape)],
        out_specs=[_row_spec(16)] * 4 + [_row_spec(32)],
        out_shape=([jax.ShapeDtypeStruct((npad, 16), jnp.float32)] * 4
                   + [jax.ShapeDtypeStruct((npad, 32), jnp.float32)]),
    )(num, hp, w, b)
    return outs[:4], outs[4]


def _pool_level(num, hp, xa, xb, bt3, wf, bf, npad):
    grid = npad // BLK
    return pl.pallas_call(
        _pool_body,
        grid=grid,
        in_specs=[_acc_spec(), _acc_spec(), _row_spec(32), _row_spec(32),
                  pl.BlockSpec((1, 1, BLK), lambda i: (i, 0, 0)),
                  _full_spec(wf.shape), _full_spec(bf.shape)],
        out_specs=pl.BlockSpec((64, 16), lambda i: (0, 0)),
        out_shape=jax.ShapeDtypeStruct((64, 16), jnp.float32),
    )(num, hp, xa, xb, bt3, wf, bf)


def _comb_final(a0, a1, a2, wc, bc):
    return pl.pallas_call(
        _comb_body,
        out_shape=jax.ShapeDtypeStruct((64, 10), jnp.float32),
    )(a0, a1, a2, wc, bc)


def _pad_edges(idx, vals, n, Epad):
    src = idx[0].astype(jnp.int32)
    dst = idx[1].astype(jnp.int32)
    vals = _f32(vals)
    E = src.shape[0]
    padi = jnp.full((Epad - E,), n, jnp.int32)
    return (jnp.concatenate([src, padi]),
            jnp.concatenate([dst, padi]),
            jnp.concatenate([vals, jnp.zeros((Epad - E,), jnp.float32)]))


@jax.jit
def kernel(X0, X1, X2, L0_indices, L0_values, L1_u_indices, L1_u_values,
           L1_d_indices, L1_d_values, L2_indices, L2_values,
           batch0, batch1, batch2, params):
    npads = [LVL[l][1] for l in range(3)]
    Xp = [_pad_rows(_f32(x), npads[i]) for i, x in enumerate((X0, X1, X2))]

    e0 = _pad_edges(L0_indices, L0_values, 10000, 327680)
    e1u = _pad_edges(L1_u_indices, L1_u_values, 10000, 327680)
    e1d = _pad_edges(L1_d_indices, L1_d_values, 10000, 327680)
    e2 = _pad_edges(L2_indices, L2_values, 5000, 163840)
    edge_args = list(e0) + list(e1u) + list(e1d) + list(e2)
    zeros_hbm = jnp.zeros((ZROWS, 16), jnp.float32)

    W1, b1 = _stage_weights(params, "s1", 128)
    W2, b2 = _stage_weights(params, "s2", 32)
    W3, b3 = _stage_weights(params, "s3", 32)

    tabs = []
    for l in range(3):
        tabs += list(_tables_s1(Xp[l], W1[l], b1[l], npads[l]))

    acc1 = _sc_call()(*tabs, *edge_args, zeros_hbm)
    num1, hp1 = acc1[0:3], acc1[3:6]

    tabs2, xs1 = [], []
    for l in range(3):
        t, xs = _combine_tables(num1[l], hp1[l], W2[l], b2[l], npads[l])
        tabs2 += list(t)
        xs1.append(xs)
    acc2 = _sc_call()(*tabs2, *edge_args, zeros_hbm)
    num2, hp2 = acc2[0:3], acc2[3:6]

    tabs3, xs2 = [], []
    for l in range(3):
        t, xs = _combine_tables(num2[l], hp2[l], W3[l], b3[l], npads[l])
        tabs3 += list(t)
        xs2.append(xs)
    acc3 = _sc_call()(*tabs3, *edge_args, zeros_hbm)
    num3, hp3 = acc3[0:3], acc3[3:6]

    wc = _f32(params["comb"]["W"])
    bc = _f32(params["comb"]["b"])[None, :]
    paccs = []
    for l, b in enumerate((batch0, batch1, batch2)):
        b = b.astype(jnp.int32)
        bt3 = jnp.concatenate(
            [b, jnp.full((npads[l] - b.shape[0],), 64, jnp.int32)]
        ).reshape(npads[l] // BLK, 1, BLK)
        wf = _f32(params[f"lvl{l}"]["final"]["W"])
        bf = _f32(params[f"lvl{l}"]["final"]["b"])[None, :]
        paccs.append(_pool_level(num3[l], hp3[l], xs1[l], xs2[l],
                                 bt3, wf, bf, npads[l]))

    return _comb_final(paccs[0], paccs[1], paccs[2], wc, bc)


# speedup vs baseline: 1.2662x; 1.2662x over previous
import functools

import jax
import jax.numpy as jnp
from jax import lax
from jax.experimental import pallas as pl
from jax.experimental.pallas import tpu as pltpu
from jax.experimental.pallas import tpu_sc as plsc

HID = 15
NSUB = 16
NCORE = 2
NW = NSUB * NCORE
CH = 128

LVL = (
    (10000, 10240, 327680, 80),
    (10000, 10240, 327680, 80),
    (5000, 5120, 163840, 40),
)
HEAD_EDGE = ((0, 0), (1, 2), (3, 3))
ZROWS = 10240 // NSUB
BLK = 640


def _f32(x):
    return x.astype(jnp.float32)


def _pad_rows(x, npad):
    return jnp.concatenate(
        [x, jnp.zeros((npad - x.shape[0], x.shape[1]), x.dtype)], axis=0)


def _head_weights(p, din_pad):
    Wl, bl, a = _f32(p["ld"]["W"]), _f32(p["ld"]["b"]), _f32(p["ld"]["a"])
    Wp, bp = _f32(p["p"]["W"]), _f32(p["p"]["b"])
    WS = jnp.concatenate([Wl, (Wl @ a[:HID])[:, None]], 1)
    bS = jnp.concatenate([bl, (bl @ a[:HID])[None]])
    WD = jnp.concatenate([Wp, (Wl @ a[HID:])[:, None]], 1)
    bD = jnp.concatenate([bp, (bl @ a[HID:])[None]])
    W = jnp.stack([WS, WD])
    if W.shape[1] < din_pad:
        W = jnp.concatenate(
            [W, jnp.zeros((2, din_pad - W.shape[1], 16), W.dtype)], 1)
    return W, jnp.stack([bS, bD])


def _stage_weights(params, stage, din_pad):
    Ws, bs = [], []
    for l in range(3):
        sp = params[f"lvl{l}"][stage]
        Wh = [_head_weights(sp[h], din_pad) for h in range(2)]
        Ws.append(jnp.stack([w for w, _ in Wh]))
        bs.append(jnp.stack([b for _, b in Wh]))
    return Ws, bs


def _tables_s1_body(x, w, b, *outs):
    xb = x[...]
    for h in range(2):
        for t in range(2):
            outs[h * 2 + t][...] = (
                jnp.dot(xb, w[h, t], preferred_element_type=jnp.float32)
                + b[h, t])


def _combine_block(num_ref, hp_ref):
    outs = []
    for h in range(2):
        num = num_ref[0, h] + num_ref[1, h]
        hp = hp_ref[0, h] + hp_ref[1, h]
        outs.append(2.0 * num[:, :HID] / (num[:, HID:HID + 1] + 1e-16)
                    + hp[:, :HID])
    xs = jax.nn.relu(jnp.concatenate(outs, 1))
    return jnp.concatenate(
        [xs, jnp.zeros((xs.shape[0], 2), xs.dtype)], 1)


def _combine_tables_body(num, hp, w, b, *outs):
    xsp = _combine_block(num, hp)
    outs[4][...] = xsp
    for h in range(2):
        for t in range(2):
            outs[h * 2 + t][...] = (
                jnp.dot(xsp, w[h, t], preferred_element_type=jnp.float32)
                + b[h, t])


def _pool_body(num, hp, xa, xb, bt, wf, bf, acc):
    i = pl.program_id(0)

    @pl.when(i == 0)
    def _():
        acc[...] = jnp.zeros_like(acc)

    xs3 = _combine_block(num, hp)
    xcat = jnp.concatenate([xa[...][:, :30], xb[...][:, :30], xs3[:, :30]], 1)
    x4 = jnp.dot(xcat, wf[...], preferred_element_type=jnp.float32) + bf[...]
    blk = x4.shape[0]
    x4e = jnp.concatenate(
        [x4, jnp.ones((blk, 1), jnp.float32), jnp.zeros((blk, 5), jnp.float32)],
        1)
    oh = (bt[0, 0, :][:, None]
          == lax.broadcasted_iota(jnp.int32, (blk, 64), 1)).astype(jnp.float32)
    acc[...] += lax.dot_general(oh, x4e, (((0,), (0,)), ((), ())),
                                preferred_element_type=jnp.float32)


def _comb_body(a0, a1, a2, wc, bc, out):
    pooled = []
    for a in (a0, a1, a2):
        acc = a[...]
        pooled.append(acc[:, :10] / jnp.maximum(acc[:, 10:11], 1.0))
    z = jnp.dot(jnp.concatenate(pooled, 1), wc[...],
                preferred_element_type=jnp.float32) + bc[...]
    z = z - jnp.max(z, axis=1, keepdims=True)
    e = jnp.exp(z)
    out[...] = e / jnp.sum(e, axis=1, keepdims=True)


def _sc_body(*refs):
    tabs = refs[0:12]
    edges = refs[12:24]
    zeros_hbm = refs[24]
    numouts = refs[25:28]
    hpouts = refs[28:31]
    (acc_num, acc_hp, srows0, drows0, srows1, drows1,
     srcidx0, dstidx0, valsv0, srcidx1, dstidx1, valsv1,
     sem_s0, sem_d0, sem_s1, sem_d1) = refs[31:]
    bufs = ((srcidx0, dstidx0, valsv0, srows0, drows0, sem_s0, sem_d0),
            (srcidx1, dstidx1, valsv1, srows1, drows1, sem_s1, sem_d1))

    c = lax.axis_index("c")
    s = lax.axis_index("s")
    wid = c * NSUB + s
    lane = lax.iota(jnp.int32, 16)
    is15 = lane == 15
    c15 = jnp.full((16,), 15, jnp.int32)

    for l, (n, npad, Epad, K) in enumerate(LVL):
        nr = npad // NSUB
        r0 = s * nr
        for h in range(2):
            S_hbm = tabs[l * 4 + h * 2 + 0]
            D_hbm = tabs[l * 4 + h * 2 + 1]
            e = HEAD_EDGE[l][h]
            src_h, dst_h, val_h = edges[e * 3], edges[e * 3 + 1], edges[e * 3 + 2]

            pltpu.sync_copy(zeros_hbm.at[pl.ds(0, nr)], acc_num.at[pl.ds(r0, nr)])
            pltpu.sync_copy(zeros_hbm.at[pl.ds(0, nr)], acc_hp.at[pl.ds(r0, nr)])
            plsc.subcore_barrier()

            def fetch_start(j, b, src_h=src_h, dst_h=dst_h, val_h=val_h,
                            S_hbm=S_hbm, D_hbm=D_hbm, K=K):
                si, di, vv, sr, dr, ss, sd = bufs[b][:7]
                off = wid * (K * CH) + j * CH
                pltpu.sync_copy(src_h.at[pl.ds(off, CH)], si)
                pltpu.sync_copy(dst_h.at[pl.ds(off, CH)], di)
                pltpu.sync_copy(val_h.at[pl.ds(off, CH)], vv)
                pltpu.async_copy(S_hbm.at[si], sr, ss)
                pltpu.async_copy(D_hbm.at[di], dr, sd)

            def process(b, S_hbm=S_hbm, D_hbm=D_hbm):
                si, di, vv_ref, sr, dr, ss, sd = bufs[b]
                pltpu.make_async_copy(S_hbm.at[si], sr, ss).wait()
                pltpu.make_async_copy(D_hbm.at[di], dr, sd).wait()

                def grpbody(g, _):
                    vv = vv_ref[pl.ds(g * 16, 16)]
                    for jj in range(16):
                        k = g * 16 + jj
                        srow = sr[k]
                        drow = dr[k]
                        z = srow + drow
                        zb = z.at[c15].get(mode="promise_in_bounds")
                        ex = jnp.exp(jnp.maximum(zb, 0.2 * zb))
                        sr[k] = jnp.where(is15, ex, ex * srow)
                        vj = vv.at[jnp.full((16,), jj, jnp.int32)].get(
                            mode="promise_in_bounds")
                        dr[k] = vj * drow
                    return 0

                lax.fori_loop(0, CH // 16, grpbody, 0)
                pltpu.sync_copy(sr, acc_num.at[di], add=True)
                pltpu.sync_copy(dr, acc_hp.at[si], add=True)

            fetch_start(0, 0)

            def pairbody(p, _, K=K, fetch_start=fetch_start, process=process):
                j0 = 2 * p
                fetch_start(j0 + 1, 1)
                process(0)

                @pl.when(j0 + 2 < K)
                def _():
                    fetch_start(j0 + 2, 0)

                process(1)
                return 0

            lax.fori_loop(0, K // 2, pairbody, 0)
            plsc.subcore_barrier()
            pltpu.sync_copy(acc_num.at[pl.ds(r0, nr)],
                            numouts[l].at[c, h, pl.ds(r0, nr)])
            pltpu.sync_copy(acc_hp.at[pl.ds(r0, nr)],
                            hpouts[l].at[c, h, pl.ds(r0, nr)])
            plsc.subcore_barrier()


def _make_sc_call():
    mesh = plsc.VectorSubcoreMesh(core_axis_name="c", subcore_axis_name="s")
    out_type = ([jax.ShapeDtypeStruct((2, 2, npad, 16), jnp.float32)
                 for (_, npad, _, _) in LVL]
                + [jax.ShapeDtypeStruct((2, 2, npad, 16), jnp.float32)
                   for (_, npad, _, _) in LVL])
    scratch = [
        pltpu.VMEM_SHARED((10240, 16), jnp.float32),
        pltpu.VMEM_SHARED((10240, 16), jnp.float32),
        pltpu.VMEM((CH, 16), jnp.float32),
        pltpu.VMEM((CH, 16), jnp.float32),
        pltpu.VMEM((CH, 16), jnp.float32),
        pltpu.VMEM((CH, 16), jnp.float32),
        pltpu.VMEM((CH,), jnp.int32),
        pltpu.VMEM((CH,), jnp.int32),
        pltpu.VMEM((CH,), jnp.float32),
        pltpu.VMEM((CH,), jnp.int32),
        pltpu.VMEM((CH,), jnp.int32),
        pltpu.VMEM((CH,), jnp.float32),
        pltpu.SemaphoreType.DMA,
        pltpu.SemaphoreType.DMA,
        pltpu.SemaphoreType.DMA,
        pltpu.SemaphoreType.DMA,
    ]
    return pl.kernel(_sc_body, out_type=out_type, mesh=mesh,
                     scratch_types=scratch,
                     compiler_params=pltpu.CompilerParams(
                         use_tc_tiling_on_sc=False))


_SC_CALL = None


def _sc_call():
    global _SC_CALL
    if _SC_CALL is None:
        _SC_CALL = _make_sc_call()
    return _SC_CALL


def _full_spec(shape):
    return pl.BlockSpec(shape, lambda i: tuple(0 for _ in shape))


def _row_spec(cols):
    return pl.BlockSpec((BLK, cols), lambda i: (i, 0))


def _acc_spec():
    return pl.BlockSpec((2, 2, BLK, 16), lambda i: (0, 0, i, 0))


def _tables_s1(xp, w, b, npad):
    grid = npad // BLK
    return pl.pallas_call(
        _tables_s1_body,
        grid=grid,
        in_specs=[_row_spec(128), _full_spec(w.shape), _full_spec(b.shape)],
        out_specs=[_row_spec(16)] * 4,
        out_shape=[jax.ShapeDtypeStruct((npad, 16), jnp.float32)] * 4,
    )(xp, w, b)


def _combine_tables(num, hp, w, b, npad):
    grid = npad // BLK
    outs = pl.pallas_call(
        _combine_tables_body,
        grid=grid,
        in_specs=[_acc_spec(), _acc_spec(),
                  _full_spec(w.shape), _full_spec(b.shape)],
        out_specs=[_row_spec(16)] * 4 + [_row_spec(32)],
        out_shape=([jax.ShapeDtypeStruct((npad, 16), jnp.float32)] * 4
                   + [jax.ShapeDtypeStruct((npad, 32), jnp.float32)]),
    )(num, hp, w, b)
    return outs[:4], outs[4]


def _pool_level(num, hp, xa, xb, bt3, wf, bf, npad):
    grid = npad // BLK
    return pl.pallas_call(
        _pool_body,
        grid=grid,
        in_specs=[_acc_spec(), _acc_spec(), _row_spec(32), _row_spec(32),
                  pl.BlockSpec((1, 1, BLK), lambda i: (i, 0, 0)),
                  _full_spec(wf.shape), _full_spec(bf.shape)],
        out_specs=pl.BlockSpec((64, 16), lambda i: (0, 0)),
        out_shape=jax.ShapeDtypeStruct((64, 16), jnp.float32),
    )(num, hp, xa, xb, bt3, wf, bf)


def _comb_final(a0, a1, a2, wc, bc):
    return pl.pallas_call(
        _comb_body,
        out_shape=jax.ShapeDtypeStruct((64, 10), jnp.float32),
    )(a0, a1, a2, wc, bc)


def _pad_edges(idx, vals, n, Epad):
    src = idx[0].astype(jnp.int32)
    dst = idx[1].astype(jnp.int32)
    vals = _f32(vals)
    E = src.shape[0]
    padi = jnp.full((Epad - E,), n, jnp.int32)
    return (jnp.concatenate([src, padi]),
            jnp.concatenate([dst, padi]),
            jnp.concatenate([vals, jnp.zeros((Epad - E,), jnp.float32)]))


@jax.jit
def kernel(X0, X1, X2, L0_indices, L0_values, L1_u_indices, L1_u_values,
           L1_d_indices, L1_d_values, L2_indices, L2_values,
           batch0, batch1, batch2, params):
    npads = [LVL[l][1] for l in range(3)]
    Xp = [_pad_rows(_f32(x), npads[i]) for i, x in enumerate((X0, X1, X2))]

    e0 = _pad_edges(L0_indices, L0_values, 10000, 327680)
    e1u = _pad_edges(L1_u_indices, L1_u_values, 10000, 327680)
    e1d = _pad_edges(L1_d_indices, L1_d_values, 10000, 327680)
    e2 = _pad_edges(L2_indices, L2_values, 5000, 163840)
    edge_args = list(e0) + list(e1u) + list(e1d) + list(e2)
    zeros_hbm = jnp.zeros((ZROWS, 16), jnp.float32)

    W1, b1 = _stage_weights(params, "s1", 128)
    W2, b2 = _stage_weights(params, "s2", 32)
    W3, b3 = _stage_weights(params, "s3", 32)

    tabs = []
    for l in range(3):
        tabs += list(_tables_s1(Xp[l], W1[l], b1[l], npads[l]))

    acc1 = _sc_call()(*tabs, *edge_args, zeros_hbm)
    num1, hp1 = acc1[0:3], acc1[3:6]

    tabs2, xs1 = [], []
    for l in range(3):
        t, xs = _combine_tables(num1[l], hp1[l], W2[l], b2[l], npads[l])
        tabs2 += list(t)
        xs1.append(xs)
    acc2 = _sc_call()(*tabs2, *edge_args, zeros_hbm)
    num2, hp2 = acc2[0:3], acc2[3:6]

    tabs3, xs2 = [], []
    for l in range(3):
        t, xs = _combine_tables(num2[l], hp2[l], W3[l], b3[l], npads[l])
        tabs3 += list(t)
        xs2.append(xs)
    acc3 = _sc_call()(*tabs3, *edge_args, zeros_hbm)
    num3, hp3 = acc3[0:3], acc3[3:6]

    wc = _f32(params["comb"]["W"])
    bc = _f32(params["comb"]["b"])[None, :]
    paccs = []
    for l, b in enumerate((batch0, batch1, batch2)):
        b = b.astype(jnp.int32)
        bt3 = jnp.concatenate(
            [b, jnp.full((npads[l] - b.shape[0],), 64, jnp.int32)]
        ).reshape(npads[l] // BLK, 1, BLK)
        wf = _f32(params[f"lvl{l}"]["final"]["W"])
        bf = _f32(params[f"lvl{l}"]["final"]["b"])[None, :]
        paccs.append(_pool_level(num3[l], hp3[l], xs1[l], xs2[l],
                                 bt3, wf, bf, npads[l]))

    return _comb_final(paccs[0], paccs[1], paccs[2], wc, bc)
